# Initial kernel scaffold; baseline (speedup 1.0000x reference)
import jax
import jax.numpy as jnp
from jax.experimental import pallas as pl


def kernel(videos_features, grid_t, grid_h, grid_w):
    raise NotImplementedError("write your pallas kernel here")


# speedup vs baseline: 2.3168x; 2.3168x over previous
import functools
import math

import jax
import jax.numpy as jnp
from jax import lax
from jax.experimental import pallas as pl
from jax.experimental.pallas import tpu as pltpu
from jax.experimental.pallas import tpu_sc as plsc

T, HW, D = 64, 1024, 1024
TOTAL = T * HW
K = max(int(math.ceil(TOTAL * 0.5)), HW)


def _scores_body(x_ref, out_ref, prev_ref):
    t = pl.program_id(0)
    x = x_ref[0]
    norm = jnp.sqrt(jnp.sum(x * x, axis=-1, keepdims=True))
    xn = x / (norm + 1e-12)
    sim = jnp.sum(xn * prev_ref[...], axis=-1, keepdims=True)
    dissim = 1.0 - sim
    out_ref[0] = jnp.where(t == 0, jnp.float32(2.0), dissim)
    prev_ref[...] = xn


def _compute_scores(videos_features):
    return pl.pallas_call(
        _scores_body,
        grid=(T,),
        in_specs=[pl.BlockSpec((1, HW, D), lambda t: (t, 0, 0))],
        out_specs=pl.BlockSpec((1, HW, 1), lambda t: (t, 0, 0)),
        out_shape=jax.ShapeDtypeStruct((T, HW, 1), jnp.float32),
        scratch_shapes=[pltpu.VMEM((HW, D), jnp.float32)],
    )(videos_features)


ROWS = TOTAL // 128


def _select_body(bits_ref, keep_ref, cnt_ref, dst_ref):
    bits = bits_ref[...]

    def bs_step(_, lohi):
        lo, hi = lohi
        mid = lo + (hi - lo + 1) // 2
        c = jnp.sum((bits >= mid).astype(jnp.int32))
        return jnp.where(c >= K, mid, lo), jnp.where(c >= K, hi, mid - 1)

    lo, _ = lax.fori_loop(0, 31, bs_step, (jnp.int32(0), jnp.int32(0x7F7FFFFF)))
    tau = lo
    gt = bits > tau
    eq = bits == tau
    need = (K - jnp.sum(gt.astype(jnp.int32))).astype(jnp.float32)

    eqf = eq.astype(jnp.float32)
    r = lax.broadcasted_iota(jnp.int32, (ROWS, ROWS), 0)
    c = lax.broadcasted_iota(jnp.int32, (ROWS, ROWS), 1)
    lower_r = (c < r).astype(jnp.float32)
    li = lax.broadcasted_iota(jnp.int32, (128, 128), 0)
    lj = lax.broadcasted_iota(jnp.int32, (128, 128), 1)
    lower_l = (li < lj).astype(jnp.float32)
    row_eq = jnp.dot(eqf, jnp.ones((128, 1), jnp.float32),
                     preferred_element_type=jnp.float32, precision=lax.Precision.HIGHEST)
    row_excl = jnp.dot(lower_r, row_eq, preferred_element_type=jnp.float32, precision=lax.Precision.HIGHEST)
    lane_excl = jnp.dot(eqf, lower_l, preferred_element_type=jnp.float32, precision=lax.Precision.HIGHEST)
    rank = row_excl + lane_excl
    keep = gt | (eq & (rank < need))
    keep_ref[...] = keep.astype(jnp.int32)

    keepf = keep.astype(jnp.float32)
    row_keep = jnp.dot(keepf, jnp.ones((128, 1), jnp.float32),
                       preferred_element_type=jnp.float32, precision=lax.Precision.HIGHEST)
    fr = lax.broadcasted_iota(jnp.int32, (T, ROWS), 0)
    rr = lax.broadcasted_iota(jnp.int32, (T, ROWS), 1)
    sel = (rr // (HW // 128) == fr).astype(jnp.float32)
    counts = jnp.dot(sel, row_keep, preferred_element_type=jnp.float32, precision=lax.Precision.HIGHEST)
    tr = lax.broadcasted_iota(jnp.int32, (T, T), 0)
    tc = lax.broadcasted_iota(jnp.int32, (T, T), 1)
    lower_t = (tc < tr).astype(jnp.float32)
    offs = jnp.dot(lower_t, counts, preferred_element_type=jnp.float32, precision=lax.Precision.HIGHEST)
    cnt_ref[...] = counts.astype(jnp.int32)

    selT_r = lax.broadcasted_iota(jnp.int32, (ROWS, T), 0)
    selT_t = lax.broadcasted_iota(jnp.int32, (ROWS, T), 1)
    selT = (selT_r // (HW // 128) == selT_t).astype(jnp.float32)
    c_row = jnp.dot(selT, counts, preferred_element_type=jnp.float32, precision=lax.Precision.HIGHEST)
    o_row = jnp.dot(selT, offs, preferred_element_type=jnp.float32, precision=lax.Precision.HIGHEST)
    rpos = lax.broadcasted_iota(jnp.int32, (ROWS, 128), 0) % (HW // 128)
    lane = lax.broadcasted_iota(jnp.int32, (ROWS, 128), 1)
    j = rpos * 128 + lane
    dst = o_row.astype(jnp.int32) + jnp.minimum(j, c_row.astype(jnp.int32) - 1)
    dst_ref[...] = dst


def _select(scores_flat_bits):
    return pl.pallas_call(
        _select_body,
        out_shape=(
            jax.ShapeDtypeStruct((ROWS, 128), jnp.int32),
            jax.ShapeDtypeStruct((T, 1), jnp.int32),
            jax.ShapeDtypeStruct((ROWS, 128), jnp.int32),
        ),
    )(scores_flat_bits)


CH = 64
NCH = HW // CH
FRAMES_PER_TILE = 2


def _sc_body(table_hbm, keep_hbm, dst_hbm, out_hbm,
             maskv, srcbuf, dstbuf, rowbuf, sem_g, sem_s):
    wid = lax.axis_index("s") * 2 + lax.axis_index("c")
    iota = lax.iota(jnp.int32, 16)
    zeros = jnp.zeros((16,), jnp.int32)
    for j in range((HW + CH) // 16):
        srcbuf[pl.ds(j * 16, 16)] = zeros

    for fi in range(FRAMES_PER_TILE):
        t = wid * FRAMES_PER_TILE + fi
        pltpu.sync_copy(keep_hbm.at[pl.ds(t * HW, HW)], maskv)
        pltpu.sync_copy(dst_hbm.at[pl.ds(t * NCH, NCH)], dstbuf)

        def compact(j, ptr):
            m = maskv[pl.ds(j * 16, 16)] != 0
            src = jnp.full((16,), t * HW + j * 16, jnp.int32) + iota
            plsc.store_compressed(srcbuf.at[pl.ds(ptr, 16)], src, mask=m)
            return ptr + jnp.sum(m.astype(jnp.int32), axis=0)

        c = lax.fori_loop(0, HW // 16, compact, jnp.int32(0))

        @pl.when(c > 0)
        def _():
            v = srcbuf[pl.ds(c - 1, 16)]
            last = jnp.sum(jnp.where(iota == 0, v, 0), axis=0)
            tailv = jnp.full((16,), 0, jnp.int32) + last
            for jj in range(CH // 16):
                srcbuf[pl.ds(c + jj * 16, 16)] = tailv

            def move(cc, carry):
                pltpu.async_copy(table_hbm.at[srcbuf.at[pl.ds(cc * CH, CH)]],
                                 rowbuf, sem_g).wait()
                pltpu.async_copy(rowbuf, out_hbm.at[dstbuf.at[cc]], sem_s).wait()
                return carry

            nch = (c + CH - 1) // CH
            lax.fori_loop(0, nch, move, jnp.int32(0))


def _sc_gather(table, keep_flat, dst2d):
    mesh = plsc.VectorSubcoreMesh(core_axis_name="c", subcore_axis_name="s")
    return pl.kernel(
        _sc_body,
        out_type=jax.ShapeDtypeStruct((K, D), jnp.float32),
        mesh=mesh,
        compiler_params=pltpu.CompilerParams(needs_layout_passes=False),
        scratch_types=[
            pltpu.VMEM((HW,), jnp.int32),
            pltpu.VMEM((HW + CH,), jnp.int32),
            pltpu.VMEM((NCH, CH), jnp.int32),
            pltpu.VMEM((CH, D), jnp.float32),
            pltpu.SemaphoreType.DMA,
            pltpu.SemaphoreType.DMA,
        ],
    )(table, keep_flat, dst2d)


def kernel(videos_features, grid_t, grid_h, grid_w):
    scores = _compute_scores(videos_features)
    bits = lax.bitcast_convert_type(
        scores.reshape(TOTAL // 128, 128), jnp.int32)
    bits = jnp.where(bits < 0, bits ^ jnp.int32(0x7FFFFFFF), bits)
    keep, counts, dst = _select(bits)
    table = videos_features.reshape(TOTAL, D)
    preserved = _sc_gather(table, keep.reshape(TOTAL), dst.reshape(T * NCH, CH))
    num_tokens = counts.reshape(T).astype(jnp.int64)
    return preserved, num_tokens
